# baseline (speedup 1.0000x reference)
import functools

import jax
import jax.numpy as jnp
from jax import lax
from jax.experimental import pallas as pl
from jax.experimental.pallas import tpu as pltpu
from jax.experimental.pallas import tpu_sc as plsc

K = 8192
D = 32
N = 4096
BETA = 0.25
DECAY = 0.99
EPS = 1e-05

NBLK = 1024

NW = 16
RPW = N // NW
CPW = K // NW
EPW = RPW * D
CEPW = CPW * D


def _argmin_body(z_ref, e_ref, idx_out, zt_out):
    zt = z_ref[...].reshape(D, NBLK)
    zt_out[...] = zt
    e2 = e_ref[...] * (-2.0)
    sz = jnp.sum(zt * zt, axis=0).reshape(NBLK, 1)
    se = 0.25 * jnp.sum(e2 * e2, axis=1)
    mm2 = lax.dot_general(zt, e2, (((0,), (1,)), ((), ())),
                          preferred_element_type=jnp.float32)
    dist = (sz + se[None, :]) + mm2
    idx_out[...] = jnp.argmin(dist, axis=1).astype(jnp.int32)


def _argmin_indices(z_e, embedding):
    return pl.pallas_call(
        _argmin_body,
        grid=(N // NBLK,),
        in_specs=[
            pl.BlockSpec((1, D, 32, 32), lambda n: (n, 0, 0, 0)),
            pl.BlockSpec((K, D), lambda n: (0, 0)),
        ],
        out_specs=[
            pl.BlockSpec((NBLK,), lambda n: (n,)),
            pl.BlockSpec((D, NBLK), lambda n: (0, n)),
        ],
        out_shape=[
            jax.ShapeDtypeStruct((N,), jnp.int32),
            jax.ShapeDtypeStruct((D, N), jnp.float32),
        ],
    )(z_e, embedding)


def _sc_body(zt_hbm, idx_hbm, cs_hbm, w_hbm, zeros_hbm, ones_hbm,
             zqst_out, loss_out,
             z_v, idx_v, eidx_v, ones_v, dw_v, w_v, cnt_v, cs_v, inv_v,
             zq_v, acc_v, dw_s, cnt_s, emb_s, sem, sem2):
    cid = lax.axis_index("c")
    sid = lax.axis_index("s")
    on = cid == 0
    w = sid

    @pl.when(on)
    def _a():
        pltpu.sync_copy(idx_hbm.at[w], idx_v)
        z_dma = pltpu.async_copy(zt_hbm.at[:, pl.ds(w * RPW, RPW)], z_v, sem)
        pltpu.async_copy(w_hbm.at[pl.ds(w * CEPW, CEPW)], w_v, sem2)
        pltpu.async_copy(cs_hbm.at[pl.ds(w * CPW, CPW)], cs_v, sem2)
        pltpu.sync_copy(zeros_hbm.at[pl.ds(w * CEPW, CEPW)],
                        dw_s.at[pl.ds(w * CEPW, CEPW)])
        pltpu.sync_copy(zeros_hbm.at[pl.ds(w * CPW, CPW)],
                        cnt_s.at[pl.ds(w * CPW, CPW)])
        pltpu.sync_copy(ones_hbm, ones_v)

        idx32 = []
        for j in range(2):
            for q in range(8):
                idx32.append(idx_v[j, pl.ds(q * 16, 16)] * D)
        for dj in range(64):
            d = dj >> 1
            for q in range(8):
                eidx_v[dj, pl.ds(q * 16, 16)] = idx32[(dj & 1) * 8 + q] + d
        z_dma.wait()

    plsc.subcore_barrier()

    @pl.when(on)
    def _a2():
        dmas = [pltpu.async_copy(z_v.at[dj >> 1, pl.ds((dj & 1) * 128, 128)],
                                 dw_s.at[eidx_v.at[dj]], sem, add=True)
                for dj in range(64)]
        dmas += [pltpu.async_copy(ones_v, cnt_s.at[idx_v.at[j]], sem,
                                  add=True)
                 for j in range(2)]
        for dma in dmas:
            dma.wait()

    plsc.subcore_barrier()

    @pl.when(on)
    def _b():
        pltpu.sync_copy(dw_s.at[pl.ds(w * CEPW, CEPW)], dw_v)
        pltpu.sync_copy(cnt_s.at[pl.ds(w * CPW, CPW)], cnt_v)
        pltpu.make_async_copy(w_hbm.at[pl.ds(w * CEPW, CEPW)], w_v,
                              sem2).wait()
        pltpu.make_async_copy(cs_hbm.at[pl.ds(w * CPW, CPW)], cs_v,
                              sem2).wait()

        def chunk(c, _):
            cnt16 = cnt_v[pl.ds(c * 16, 16)]
            cs16 = cs_v[pl.ds(c * 16, 16)]
            n16 = (cs16 * DECAY + (1.0 - DECAY) * cnt16) + EPS
            inv_v[pl.ds(c * 16, 16)] = 1.0 / n16
            return 0

        lax.fori_loop(0, CPW // 16, chunk, 0)

        def code(r, _):
            inv = plsc.load_gather(inv_v, [jnp.full((16,), 0, jnp.int32) + r])
            for h in range(2):
                sl = pl.ds(r * D + h * 16, 16)
                new16 = (w_v[sl] * DECAY + (1.0 - DECAY) * dw_v[sl]) * inv
                dw_v[sl] = new16
            return 0

        lax.fori_loop(0, CPW, code, 0)
        pltpu.sync_copy(dw_v, emb_s.at[pl.ds(w * CEPW, CEPW)])

    plsc.subcore_barrier()

    @pl.when(on)
    def _c():
        dmas = [pltpu.async_copy(emb_s.at[eidx_v.at[dj]],
                                 zq_v.at[dj >> 1, pl.ds((dj & 1) * 128, 128)],
                                 sem)
                for dj in range(64)]
        for dma in dmas:
            dma.wait()

        acc = jnp.zeros((16,), jnp.float32)
        for d in range(D):
            def piece(c, a, d=d):
                sl = pl.ds(c * 16, 16)
                zz = z_v[d, sl]
                q = zq_v[d, sl]
                z_v[d, sl] = zz + (q - zz)
                dd = zz - q
                return a + dd * dd

            acc = lax.fori_loop(0, RPW // 16, piece, acc)
        acc_v[...] = acc
        pltpu.sync_copy(z_v, zqst_out.at[w >> 2, :,
                                         pl.ds((w & 3) * RPW, RPW)])
        pltpu.sync_copy(acc_v, loss_out.at[pl.ds(w * 16, 16)])


@functools.partial(
    pl.kernel,
    out_type=(
        jax.ShapeDtypeStruct((4, D, 1024), jnp.float32),
        jax.ShapeDtypeStruct((NW * 16,), jnp.float32),
    ),
    mesh=plsc.VectorSubcoreMesh(core_axis_name="c", subcore_axis_name="s"),
    compiler_params=pltpu.CompilerParams(needs_layout_passes=False),
    scratch_types=[
        pltpu.VMEM((D, RPW), jnp.float32),
        pltpu.VMEM((2, 128), jnp.int32),
        pltpu.VMEM((64, 128), jnp.int32),
        pltpu.VMEM((128,), jnp.float32),
        pltpu.VMEM((CEPW,), jnp.float32),
        pltpu.VMEM((CEPW,), jnp.float32),
        pltpu.VMEM((CPW,), jnp.float32),
        pltpu.VMEM((CPW,), jnp.float32),
        pltpu.VMEM((CPW,), jnp.float32),
        pltpu.VMEM((D, RPW), jnp.float32),
        pltpu.VMEM((16,), jnp.float32),
        pltpu.VMEM_SHARED((K * D,), jnp.float32),
        pltpu.VMEM_SHARED((K,), jnp.float32),
        pltpu.VMEM_SHARED((K * D,), jnp.float32),
        pltpu.SemaphoreType.DMA,
        pltpu.SemaphoreType.DMA,
    ],
)
def _sc_update(zt_hbm, idx_hbm, cs_hbm, w_hbm, zeros_hbm, ones_hbm,
               zqst_out, loss_out, *rest):
    _sc_body(zt_hbm, idx_hbm, cs_hbm, w_hbm, zeros_hbm, ones_hbm,
             zqst_out, loss_out, *rest)


def kernel(z_e, embedding, ema_cluster_size, ema_w):
    B, Dd, H, W = z_e.shape
    indices, zt = _argmin_indices(z_e, embedding)

    zeros = jnp.zeros((K * D,), jnp.float32)
    ones = jnp.ones((128,), jnp.float32)
    zqst3, loss_part = _sc_update(
        zt, indices.reshape(NW, 2, 128), ema_cluster_size,
        ema_w.reshape(-1), zeros, ones)

    z_q_st = zqst3.reshape(B, Dd, H, W)
    vq_loss = BETA * (jnp.sum(loss_part) / (N * D))
    return (z_q_st, vq_loss, indices.reshape(B, H, W))
